# Initial kernel scaffold; baseline (speedup 1.0000x reference)
import jax
import jax.numpy as jnp
from jax.experimental import pallas as pl


def kernel(x, edge_index, W1, W2, W3):
    raise NotImplementedError("write your pallas kernel here")


# speedup vs baseline: 3.3762x; 3.3762x over previous
import functools

import jax
import jax.numpy as jnp
from jax import lax
from jax.experimental import pallas as pl
from jax.experimental.pallas import tpu as pltpu
from jax.experimental.pallas import tpu_sc as plsc

N = 10000
D = 128
E = 320000

NC = 2
NS = 16
NW = NC * NS

CHUNK = 128
NCH = 80
HALF = NCH // 2
E_PAD = NW * NCH * CHUNK
TPR = 632
ACC_N = NS * TPR
DUMMY = N

ROW_BLK = 2000


def _sc_mesh():
  return plsc.VectorSubcoreMesh(core_axis_name="c", subcore_axis_name="s")


def _sc_body(h_hbm, src_hbm, dst_hbm, out_hbm,
             src_v, dst_v, rows_a, rows_b, acc_sh, sem_a, sem_b):
  cid = lax.axis_index("c")
  sid = lax.axis_index("s")
  wid = cid * NS + sid

  def _zrow(i, carry):
    for j in range(D // 16):
      rows_a[i, pl.ds(j * 16, 16)] = jnp.zeros((16,), jnp.float32)
    return carry
  lax.fori_loop(0, CHUNK, _zrow, 0)
  zbase = sid * TPR
  nfull = TPR // CHUNK
  for k in range(nfull):
    pltpu.sync_copy(rows_a, acc_sh.at[pl.ds(zbase + k * CHUNK, CHUNK)])
  rem = TPR - nfull * CHUNK
  if rem:
    pltpu.sync_copy(rows_a.at[pl.ds(0, rem)],
                    acc_sh.at[pl.ds(zbase + nfull * CHUNK, rem)])
  plsc.subcore_barrier()

  def _start(j, buf, sem):
    pltpu.async_copy(h_hbm.at[src_v.at[j]], buf, sem)

  def _wait(j, buf, sem):
    pltpu.make_async_copy(h_hbm.at[src_v.at[j]], buf, sem).wait()

  def _scat(j, buf):
    pltpu.sync_copy(buf, acc_sh.at[dst_v.at[j]], add=True)

  def _pipe(i, carry):
    j = i * 2
    _start(j + 1, rows_b, sem_b)
    _wait(j, rows_a, sem_a)
    _scat(j, rows_a)

    @pl.when(j + 2 < HALF)
    def _():
      _start(j + 2, rows_a, sem_a)

    _wait(j + 1, rows_b, sem_b)
    _scat(j + 1, rows_b)
    return carry

  for half in range(NCH // HALF):
    pltpu.sync_copy(src_hbm.at[wid, pl.ds(half * HALF, HALF)], src_v)
    pltpu.sync_copy(dst_hbm.at[wid, pl.ds(half * HALF, HALF)], dst_v)
    _start(0, rows_a, sem_a)
    lax.fori_loop(0, HALF // 2, _pipe, 0)
  plsc.subcore_barrier()

  pltpu.sync_copy(acc_sh.at[pl.ds(sid * TPR, TPR)],
                  out_hbm.at[cid, pl.ds(sid * TPR, TPR)])


def _sc_aggregate(h, src_r, dst_r):
  kern = pl.kernel(
      _sc_body,
      out_type=jax.ShapeDtypeStruct((NC, ACC_N, D), jnp.float32),
      mesh=_sc_mesh(),
      scratch_types=[
          pltpu.VMEM((HALF, CHUNK), jnp.int32),
          pltpu.VMEM((HALF, CHUNK), jnp.int32),
          pltpu.VMEM((CHUNK, D), jnp.float32),
          pltpu.VMEM((CHUNK, D), jnp.float32),
          pltpu.VMEM_SHARED((ACC_N, D), jnp.float32),
          pltpu.SemaphoreType.DMA,
          pltpu.SemaphoreType.DMA,
      ],
  )
  return kern(h, src_r, dst_r)


def _mm_body(x_ref, w_ref, o_ref):
  o_ref[...] = jnp.dot(x_ref[...], w_ref[...],
                       preferred_element_type=jnp.float32)


def _tc_matmul(x, w):
  return pl.pallas_call(
      _mm_body,
      grid=(N // ROW_BLK,),
      in_specs=[
          pl.BlockSpec((ROW_BLK, D), lambda i: (i, 0)),
          pl.BlockSpec((D, D), lambda i: (0, 0)),
      ],
      out_specs=pl.BlockSpec((ROW_BLK, D), lambda i: (i, 0)),
      out_shape=jax.ShapeDtypeStruct((N, D), jnp.float32),
  )(x, w)


def _comb_body(p0_ref, p1_ref, x_ref, w_ref, xn_ref, h_ref):
  xn = x_ref[...] + jnp.maximum(p0_ref[...] + p1_ref[...], 0.0)
  xn_ref[...] = xn
  h_ref[...] = jnp.dot(xn, w_ref[...], preferred_element_type=jnp.float32)


def _tc_combine(p, x, w):
  blk = pl.BlockSpec((ROW_BLK, D), lambda i: (i, 0))
  return pl.pallas_call(
      _comb_body,
      grid=(N // ROW_BLK,),
      in_specs=[blk, blk, blk, pl.BlockSpec((D, D), lambda i: (0, 0))],
      out_specs=[blk, blk],
      out_shape=[
          jax.ShapeDtypeStruct((N, D), jnp.float32),
          jax.ShapeDtypeStruct((N, D), jnp.float32),
      ],
  )(p[0], p[1], x, w)


def _relu_body(p0_ref, p1_ref, o_ref):
  o_ref[...] = jnp.maximum(p0_ref[...] + p1_ref[...], 0.0)


def _tc_final(p):
  blk = pl.BlockSpec((ROW_BLK, D), lambda i: (i, 0))
  return pl.pallas_call(
      _relu_body,
      grid=(N // ROW_BLK,),
      in_specs=[blk, blk],
      out_specs=blk,
      out_shape=jax.ShapeDtypeStruct((N, D), jnp.float32),
  )(p[0], p[1])


@jax.jit
def kernel(x, edge_index, W1, W2, W3):
  src = edge_index[0].astype(jnp.int32)
  dst = edge_index[1].astype(jnp.int32)
  pad = E_PAD - E
  src_r = jnp.concatenate(
      [src, jnp.zeros((pad,), jnp.int32)]).reshape(NW, NCH, CHUNK)
  dst_r = jnp.concatenate(
      [dst, jnp.full((pad,), DUMMY, jnp.int32)]).reshape(NW, NCH, CHUNK)

  h1 = _tc_matmul(x, W1)
  p1 = _sc_aggregate(h1, src_r, dst_r)
  x2, h2 = _tc_combine(p1, x, W2)
  p2 = _sc_aggregate(h2, src_r, dst_r)
  x3, h3 = _tc_combine(p2, x2, W3)
  p3 = _sc_aggregate(h3, src_r, dst_r)
  return _tc_final(p3)
